# baseline (device time: 26276 ns/iter reference)
import functools

import jax
import jax.numpy as jnp
from jax import lax
from jax.experimental import pallas as pl
from jax.experimental.pallas import tpu as pltpu

N_DEV = 4
N_SRC = 2
B = 2
SQ = 128
SKV_LOC = 128
H_LOC = 4
DH = 64
D_MODEL = 512
QW = D_MODEL // N_DEV
HD = H_LOC * DH

_KV_ORDER = {0: (2, 3, 1), 1: (3, 2, 0)}

_DeviceIdType = getattr(pl, "DeviceIdType", None) or pltpu.DeviceIdType
_sem_signal = getattr(pl, "semaphore_signal", None) or pltpu.semaphore_signal
_sem_wait = getattr(pl, "semaphore_wait", None) or pltpu.semaphore_wait
_CompilerParams = getattr(pltpu, "CompilerParams", None) or getattr(
    pltpu, "TPUCompilerParams"
)


def kernel(x, Wq, K_ext, V_ext, Wo):
    K_t = jnp.transpose(K_ext, (0, 2, 3, 1))
    V_t = jnp.transpose(V_ext, (0, 2, 3, 1))

    def body(x_ref, wq_ref, k_ref, v_ref, wo_ref, out_ref,
             x_v, wq_v, wo_v, kvst, out_stage,
             kv_send, kv_recv, qsend, rs_recv, ag_send, ag_recv,
             in_sems, kvst_sems, out_sem,
             kv_send_sems, kv_recv_sems,
             rs_send_sems, rs_recv_sems, ag_send_sems, ag_recv_sems):
        my = lax.axis_index("i")

        x_cp = pltpu.make_async_copy(x_ref, x_v, in_sems.at[0])
        wq_cp = pltpu.make_async_copy(wq_ref, wq_v, in_sems.at[1])
        wo_cp = pltpu.make_async_copy(wo_ref, wo_v, in_sems.at[2])
        x_cp.start()
        wq_cp.start()
        wo_cp.start()

        def _kvst_desc(t, kv, ref):
            return pltpu.make_async_copy(
                ref.at[:, pl.ds(H_LOC * t, H_LOC), :, :], kvst.at[t, kv],
                kvst_sems.at[t, kv])

        for p in range(N_SRC):
            @pl.when(my == p)
            def _(p=p):
                for t in range(N_DEV):
                    _kvst_desc(t, 0, k_ref).start()
                    _kvst_desc(t, 1, v_ref).start()

        barrier = pltpu.get_barrier_semaphore()
        for d in range(1, N_DEV):
            _sem_signal(barrier, inc=1, device_id=((my + d) % N_DEV,),
                        device_id_type=_DeviceIdType.MESH)
        _sem_wait(barrier, N_DEV - 1)

        def _kv_send_desc(p, t):
            return pltpu.make_async_remote_copy(
                src_ref=kv_send.at[t], dst_ref=kv_recv.at[p],
                send_sem=kv_send_sems.at[t], recv_sem=kv_recv_sems.at[p],
                device_id=(t,), device_id_type=_DeviceIdType.MESH)

        for p in range(N_SRC):
            @pl.when(my == p)
            def _(p=p):
                def _pack(t, kv):
                    _kvst_desc(t, kv, k_ref).wait()
                    return kvst[t, kv].reshape(B, HD, SKV_LOC).astype(
                        jnp.bfloat16)

                for t in _KV_ORDER[p]:
                    kv_send[t, 0] = _pack(t, 0)
                    kv_send[t, 1] = _pack(t, 1)
                    _kv_send_desc(p, t).start()
                kv_recv[p, 0] = _pack(p, 0)
                kv_recv[p, 1] = _pack(p, 1)

        x_cp.wait()
        wq_cp.wait()
        wqb = (wq_v[...] * 0.125).astype(jnp.bfloat16)
        qs = []
        for b in range(B):
            qs.append(lax.dot(x_v[b].astype(jnp.bfloat16), wqb,
                              preferred_element_type=jnp.float32
                              ).astype(jnp.bfloat16))

        iota_i = lax.broadcasted_iota(jnp.int32, (SQ, SKV_LOC), 0)
        iota_j = lax.broadcasted_iota(jnp.int32, (SQ, SKV_LOC), 1)
        causal = iota_j <= iota_i

        @pl.when(my != 0)
        def _():
            _kv_send_desc(0, 0).wait_recv()

        qhs = [[qs[b][:, h * DH:(h + 1) * DH]
                for h in range(H_LOC)] for b in range(B)]
        s0s = [[lax.dot(qhs[b][h],
                        kv_recv[0, 0, b][h * DH:(h + 1) * DH, :],
                        preferred_element_type=jnp.float32)
                for h in range(H_LOC)] for b in range(B)]

        @pl.when(my != 1)
        def _():
            _kv_send_desc(1, 1).wait_recv()

        def _rs_desc(s, t, b):
            return pltpu.make_async_remote_copy(
                src_ref=qsend.at[t, b], dst_ref=rs_recv.at[s, b],
                send_sem=rs_send_sems.at[t, b], recv_sem=rs_recv_sems.at[s, b],
                device_id=(t,), device_id_type=_DeviceIdType.MESH)

        def _ag_desc(s, t, b):
            return pltpu.make_async_remote_copy(
                src_ref=ag_send.at[b], dst_ref=ag_recv.at[s, b],
                send_sem=ag_send_sems.at[t, b], recv_sem=ag_recv_sems.at[s, b],
                device_id=(t,), device_id_type=_DeviceIdType.MESH)

        wo_cp.wait()
        wob = wo_v[...].astype(jnp.bfloat16)
        for b in range(B):
            k1 = kv_recv[1, 0, b]
            v0 = kv_recv[0, 1, b]
            v1 = kv_recv[1, 1, b]
            ctx_heads = []
            for h in range(H_LOC):
                hs = slice(h * DH, (h + 1) * DH)
                s1 = lax.dot(qhs[b][h], k1[hs, :],
                             preferred_element_type=jnp.float32)
                w0 = jnp.exp(s0s[b][h])
                w1 = jnp.where(causal, jnp.exp(s1), 0.0)
                l = jnp.sum(w0, axis=1, keepdims=True) + jnp.sum(
                    w1, axis=1, keepdims=True)
                c = (lax.dot_general(w0.astype(jnp.bfloat16), v0[hs, :],
                                     (((1,), (1,)), ((), ())),
                                     preferred_element_type=jnp.float32)
                     + lax.dot_general(w1.astype(jnp.bfloat16), v1[hs, :],
                                       (((1,), (1,)), ((), ())),
                                       preferred_element_type=jnp.float32)) / l
                ctx_heads.append(c)
            ctx_b = jnp.concatenate(ctx_heads, axis=1).astype(jnp.bfloat16)
            pout_b = lax.dot(ctx_b, wob, preferred_element_type=jnp.float32)
            out_stage[b] = pout_b
            for t in range(N_DEV):
                qsend[t, b] = pout_b[:, QW * t:QW * (t + 1)].astype(jnp.bfloat16)
            for s in range(N_DEV):
                @pl.when(my == s)
                def _(s=s, b=b):
                    for t in range(N_DEV):
                        if t != s:
                            _rs_desc(s, t, b).start()

        for b in range(B):
            for s in range(N_DEV):
                @pl.when(my == s)
                def _(s=s, b=b):
                    for t in range(N_DEV):
                        if t != s:
                            _rs_desc(t, s, b).wait_recv()
                    q = out_stage[b, :, QW * s:QW * (s + 1)]
                    for t in range(N_DEV):
                        if t != s:
                            q = q + rs_recv[t, b].astype(jnp.float32)
                    out_stage[b, :, QW * s:QW * (s + 1)] = q
                    ag_send[b] = q.astype(jnp.bfloat16)
                    for t in range(N_DEV):
                        if t != s:
                            _ag_desc(s, t, b).start()

        for b in range(B):
            for s in range(N_DEV):
                @pl.when(my != s)
                def _(s=s, b=b):
                    _ag_desc(s, 0, b).wait_recv()
                    out_stage[b, :, QW * s:QW * (s + 1)] = (
                        ag_recv[s, b].astype(jnp.float32))

        out_cp = pltpu.make_async_copy(out_stage, out_ref, out_sem)
        out_cp.start()

        for b in range(B):
            for s in range(N_DEV):
                @pl.when(my == s)
                def _(s=s, b=b):
                    for t in range(N_DEV):
                        if t != s:
                            _rs_desc(s, t, b).wait_send()
                            _ag_desc(s, t, b).wait_send()
        for p in range(N_SRC):
            @pl.when(my == p)
            def _(p=p):
                for t in range(N_DEV):
                    if t != p:
                        _kv_send_desc(p, t).wait_send()
        out_cp.wait()

        @functools.partial(pl.run_scoped, exit_sem=pltpu.SemaphoreType.REGULAR)
        def _(exit_sem):
            for d in range(1, N_DEV):
                _sem_signal(exit_sem, inc=1, device_id=((my + d) % N_DEV,),
                            device_id_type=_DeviceIdType.MESH)
            _sem_wait(exit_sem, N_DEV - 1)

    return pl.pallas_call(
        body,
        out_shape=jax.ShapeDtypeStruct((B, SQ, D_MODEL), jnp.float32),
        in_specs=[pl.BlockSpec(memory_space=pl.ANY)] * 5,
        out_specs=pl.BlockSpec(memory_space=pl.ANY),
        scratch_shapes=[
            pltpu.VMEM((B, SQ, D_MODEL), jnp.float32),
            pltpu.VMEM((D_MODEL, HD), jnp.float32),
            pltpu.VMEM((HD, D_MODEL), jnp.float32),
            pltpu.VMEM((N_DEV, 2, B, H_LOC, DH, SKV_LOC), jnp.float32),
            pltpu.VMEM((B, SQ, D_MODEL), jnp.float32),
            pltpu.VMEM((N_DEV, 2, B, HD, SKV_LOC), jnp.bfloat16),
            pltpu.VMEM((N_SRC, 2, B, HD, SKV_LOC), jnp.bfloat16),
            pltpu.VMEM((N_DEV, B, SQ, QW), jnp.bfloat16),
            pltpu.VMEM((N_DEV, B, SQ, QW), jnp.bfloat16),
            pltpu.VMEM((B, SQ, QW), jnp.bfloat16),
            pltpu.VMEM((N_DEV, B, SQ, QW), jnp.bfloat16),
            pltpu.SemaphoreType.DMA((3,)),
            pltpu.SemaphoreType.DMA((N_DEV, 2)),
            pltpu.SemaphoreType.DMA,
            pltpu.SemaphoreType.DMA((N_DEV,)),
            pltpu.SemaphoreType.DMA((N_SRC,)),
            pltpu.SemaphoreType.DMA((N_DEV, B)),
            pltpu.SemaphoreType.DMA((N_DEV, B)),
            pltpu.SemaphoreType.DMA((N_DEV, B)),
            pltpu.SemaphoreType.DMA((N_DEV, B)),
        ],
        compiler_params=_CompilerParams(collective_id=0),
    )(x, Wq, K_t, V_t, Wo)


# device time: 23709 ns/iter; 1.1083x vs baseline; 1.1083x over previous
import functools

import jax
import jax.numpy as jnp
from jax import lax
from jax.experimental import pallas as pl
from jax.experimental.pallas import tpu as pltpu

N_DEV = 4
N_SRC = 2
B = 2
SQ = 128
SKV_LOC = 128
H_LOC = 4
DH = 64
D_MODEL = 512
QW = D_MODEL // N_DEV
HD = H_LOC * DH

_KV_ORDER = {0: (2, 3, 1), 1: (3, 2, 0)}

_DeviceIdType = getattr(pl, "DeviceIdType", None) or pltpu.DeviceIdType
_sem_signal = getattr(pl, "semaphore_signal", None) or pltpu.semaphore_signal
_sem_wait = getattr(pl, "semaphore_wait", None) or pltpu.semaphore_wait
_CompilerParams = getattr(pltpu, "CompilerParams", None) or getattr(
    pltpu, "TPUCompilerParams"
)


def kernel(x, Wq, K_ext, V_ext, Wo):
    K_t = jnp.transpose(K_ext, (0, 2, 3, 1))
    V_t = jnp.transpose(V_ext, (0, 2, 3, 1))
    xww = jnp.concatenate(
        [x.reshape(B * SQ, D_MODEL), Wq.T, Wo], axis=0)

    def body(xww_ref, k_ref, v_ref, out_ref,
             xww_v, kvst, out_stage,
             kv_send, kv_recv, qsend, rs_recv, ag_send, ag_recv,
             in_sems, kvst_sems, out_sem,
             kv_send_sems, kv_recv_sems,
             rs_send_sems, rs_recv_sems, ag_send_sems, ag_recv_sems):
        my = lax.axis_index("i")

        xww_cp = pltpu.make_async_copy(xww_ref, xww_v, in_sems.at[0])
        xww_cp.start()

        def _kvst_desc(t, kv, ref):
            return pltpu.make_async_copy(
                ref.at[:, pl.ds(H_LOC * t, H_LOC), :, :], kvst.at[t, kv],
                kvst_sems.at[t, kv])

        for p in range(N_SRC):
            @pl.when(my == p)
            def _(p=p):
                for t in range(N_DEV):
                    _kvst_desc(t, 0, k_ref).start()
                    _kvst_desc(t, 1, v_ref).start()

        barrier = pltpu.get_barrier_semaphore()
        for d in range(1, N_DEV):
            _sem_signal(barrier, inc=1, device_id=((my + d) % N_DEV,),
                        device_id_type=_DeviceIdType.MESH)
        _sem_wait(barrier, N_DEV - 1)

        def _kv_send_desc(p, t):
            return pltpu.make_async_remote_copy(
                src_ref=kv_send.at[t], dst_ref=kv_recv.at[p],
                send_sem=kv_send_sems.at[t], recv_sem=kv_recv_sems.at[p],
                device_id=(t,), device_id_type=_DeviceIdType.MESH)

        for p in range(N_SRC):
            @pl.when(my == p)
            def _(p=p):
                def _pack(t, kv):
                    _kvst_desc(t, kv, k_ref).wait()
                    return kvst[t, kv].reshape(B, HD, SKV_LOC).astype(
                        jnp.bfloat16)

                for t in _KV_ORDER[p]:
                    kv_send[t, 0] = _pack(t, 0)
                    kv_send[t, 1] = _pack(t, 1)
                    _kv_send_desc(p, t).start()
                kv_recv[p, 0] = _pack(p, 0)
                kv_recv[p, 1] = _pack(p, 1)

        xww_cp.wait()
        wqtb = (xww_v[2 * SQ:2 * SQ + HD, :] * 0.125).astype(jnp.bfloat16)
        qs = []
        for b in range(B):
            xb = xww_v[b * SQ:(b + 1) * SQ, :].astype(jnp.bfloat16)
            qs.append(lax.dot_general(xb, wqtb, (((1,), (1,)), ((), ())),
                                      preferred_element_type=jnp.float32
                                      ).astype(jnp.bfloat16))

        iota_i = lax.broadcasted_iota(jnp.int32, (SQ, SKV_LOC), 0)
        iota_j = lax.broadcasted_iota(jnp.int32, (SQ, SKV_LOC), 1)
        causal = iota_j <= iota_i

        @pl.when(my != 0)
        def _():
            _kv_send_desc(0, 0).wait_recv()

        qhs = [[qs[b][:, h * DH:(h + 1) * DH]
                for h in range(H_LOC)] for b in range(B)]
        s0s = [[lax.dot(qhs[b][h],
                        kv_recv[0, 0, b][h * DH:(h + 1) * DH, :],
                        preferred_element_type=jnp.float32)
                for h in range(H_LOC)] for b in range(B)]

        @pl.when(my != 1)
        def _():
            _kv_send_desc(1, 1).wait_recv()

        def _rs_desc(s, t, b):
            return pltpu.make_async_remote_copy(
                src_ref=qsend.at[t, b], dst_ref=rs_recv.at[s, b],
                send_sem=rs_send_sems.at[t, b], recv_sem=rs_recv_sems.at[s, b],
                device_id=(t,), device_id_type=_DeviceIdType.MESH)

        def _ag_desc(s, t, b):
            return pltpu.make_async_remote_copy(
                src_ref=ag_send.at[b], dst_ref=ag_recv.at[s, b],
                send_sem=ag_send_sems.at[t, b], recv_sem=ag_recv_sems.at[s, b],
                device_id=(t,), device_id_type=_DeviceIdType.MESH)

        wob = xww_v[2 * SQ + HD:, :].astype(jnp.bfloat16)
        for b in range(B):
            k1 = kv_recv[1, 0, b]
            v0 = kv_recv[0, 1, b]
            v1 = kv_recv[1, 1, b]
            ctx_heads = []
            for h in range(H_LOC):
                hs = slice(h * DH, (h + 1) * DH)
                s1 = lax.dot(qhs[b][h], k1[hs, :],
                             preferred_element_type=jnp.float32)
                w0 = jnp.exp(s0s[b][h])
                w1 = jnp.where(causal, jnp.exp(s1), 0.0)
                l = jnp.sum(w0, axis=1, keepdims=True) + jnp.sum(
                    w1, axis=1, keepdims=True)
                c = (lax.dot_general(w0.astype(jnp.bfloat16), v0[hs, :],
                                     (((1,), (1,)), ((), ())),
                                     preferred_element_type=jnp.float32)
                     + lax.dot_general(w1.astype(jnp.bfloat16), v1[hs, :],
                                       (((1,), (1,)), ((), ())),
                                       preferred_element_type=jnp.float32)) / l
                ctx_heads.append(c)
            ctx_b = jnp.concatenate(ctx_heads, axis=1).astype(jnp.bfloat16)
            pout_b = lax.dot(ctx_b, wob, preferred_element_type=jnp.float32)
            out_stage[b] = pout_b
            for t in range(N_DEV):
                qsend[t, b] = pout_b[:, QW * t:QW * (t + 1)].astype(jnp.bfloat16)
            for s in range(N_DEV):
                @pl.when(my == s)
                def _(s=s, b=b):
                    for t in range(N_DEV):
                        if t != s:
                            _rs_desc(s, t, b).start()

        for b in range(B):
            for s in range(N_DEV):
                @pl.when(my == s)
                def _(s=s, b=b):
                    for t in range(N_DEV):
                        if t != s:
                            _rs_desc(t, s, b).wait_recv()
                    q = out_stage[b, :, QW * s:QW * (s + 1)]
                    for t in range(N_DEV):
                        if t != s:
                            q = q + rs_recv[t, b].astype(jnp.float32)
                    out_stage[b, :, QW * s:QW * (s + 1)] = q
                    ag_send[b] = q.astype(jnp.bfloat16)
                    for t in range(N_DEV):
                        if t != s:
                            _ag_desc(s, t, b).start()

        for b in range(B):
            for s in range(N_DEV):
                @pl.when(my != s)
                def _(s=s, b=b):
                    _ag_desc(s, 0, b).wait_recv()
                    out_stage[b, :, QW * s:QW * (s + 1)] = (
                        ag_recv[s, b].astype(jnp.float32))

        out_cp = pltpu.make_async_copy(out_stage, out_ref, out_sem)
        out_cp.start()

        for b in range(B):
            for s in range(N_DEV):
                @pl.when(my == s)
                def _(s=s, b=b):
                    for t in range(N_DEV):
                        if t != s:
                            _rs_desc(s, t, b).wait_send()
                            _ag_desc(s, t, b).wait_send()
        for p in range(N_SRC):
            @pl.when(my == p)
            def _(p=p):
                for t in range(N_DEV):
                    if t != p:
                        _kv_send_desc(p, t).wait_send()
        out_cp.wait()

        @functools.partial(pl.run_scoped, exit_sem=pltpu.SemaphoreType.REGULAR)
        def _(exit_sem):
            for d in range(1, N_DEV):
                _sem_signal(exit_sem, inc=1, device_id=((my + d) % N_DEV,),
                            device_id_type=_DeviceIdType.MESH)
            _sem_wait(exit_sem, N_DEV - 1)

    return pl.pallas_call(
        body,
        out_shape=jax.ShapeDtypeStruct((B, SQ, D_MODEL), jnp.float32),
        in_specs=[pl.BlockSpec(memory_space=pl.ANY)] * 3,
        out_specs=pl.BlockSpec(memory_space=pl.ANY),
        scratch_shapes=[
            pltpu.VMEM((2 * SQ + HD + HD, D_MODEL), jnp.float32),
            pltpu.VMEM((N_DEV, 2, B, H_LOC, DH, SKV_LOC), jnp.float32),
            pltpu.VMEM((B, SQ, D_MODEL), jnp.float32),
            pltpu.VMEM((N_DEV, 2, B, HD, SKV_LOC), jnp.bfloat16),
            pltpu.VMEM((N_SRC, 2, B, HD, SKV_LOC), jnp.bfloat16),
            pltpu.VMEM((N_DEV, B, SQ, QW), jnp.bfloat16),
            pltpu.VMEM((N_DEV, B, SQ, QW), jnp.bfloat16),
            pltpu.VMEM((B, SQ, QW), jnp.bfloat16),
            pltpu.VMEM((N_DEV, B, SQ, QW), jnp.bfloat16),
            pltpu.SemaphoreType.DMA((1,)),
            pltpu.SemaphoreType.DMA((N_DEV, 2)),
            pltpu.SemaphoreType.DMA,
            pltpu.SemaphoreType.DMA((N_DEV,)),
            pltpu.SemaphoreType.DMA((N_SRC,)),
            pltpu.SemaphoreType.DMA((N_DEV, B)),
            pltpu.SemaphoreType.DMA((N_DEV, B)),
            pltpu.SemaphoreType.DMA((N_DEV, B)),
            pltpu.SemaphoreType.DMA((N_DEV, B)),
        ],
        compiler_params=_CompilerParams(collective_id=0),
    )(xww, K_t, V_t)


# device time: 23607 ns/iter; 1.1131x vs baseline; 1.0043x over previous
import functools

import jax
import jax.numpy as jnp
from jax import lax
from jax.experimental import pallas as pl
from jax.experimental.pallas import tpu as pltpu

N_DEV = 4
N_SRC = 2
B = 2
SQ = 128
SKV_LOC = 128
H_LOC = 4
DH = 64
D_MODEL = 512
QW = D_MODEL // N_DEV
HD = H_LOC * DH

_KV_ORDER = {0: (2, 3, 1), 1: (3, 2, 0)}

_DeviceIdType = getattr(pl, "DeviceIdType", None) or pltpu.DeviceIdType
_sem_signal = getattr(pl, "semaphore_signal", None) or pltpu.semaphore_signal
_sem_wait = getattr(pl, "semaphore_wait", None) or pltpu.semaphore_wait
_CompilerParams = getattr(pltpu, "CompilerParams", None) or getattr(
    pltpu, "TPUCompilerParams"
)


def kernel(x, Wq, K_ext, V_ext, Wo):
    K_t = jnp.transpose(K_ext, (0, 2, 3, 1))
    V_t = jnp.transpose(V_ext, (0, 2, 3, 1))
    xww = jnp.concatenate(
        [x.reshape(B * SQ, D_MODEL), Wq.T, Wo], axis=0)

    def body(xww_ref, k_ref, v_ref, out_ref,
             kvst, out_stage,
             kv_send, kv_recv, qsend, rs_recv, ag_send, ag_recv,
             kvst_sems, out_sem,
             kv_send_sems, kv_recv_sems,
             rs_send_sems, rs_recv_sems, ag_send_sems, ag_recv_sems):
        my = lax.axis_index("i")
        xww_v = xww_ref

        def _kvst_desc(t, kv, ref):
            return pltpu.make_async_copy(
                ref.at[:, pl.ds(H_LOC * t, H_LOC), :, :], kvst.at[t, kv],
                kvst_sems.at[t, kv])

        for p in range(N_SRC):
            @pl.when(my == p)
            def _(p=p):
                for t in range(N_DEV):
                    _kvst_desc(t, 0, k_ref).start()
                    _kvst_desc(t, 1, v_ref).start()

        barrier = pltpu.get_barrier_semaphore()
        for d in range(1, N_DEV):
            _sem_signal(barrier, inc=1, device_id=((my + d) % N_DEV,),
                        device_id_type=_DeviceIdType.MESH)
        _sem_wait(barrier, N_DEV - 1)

        def _kv_send_desc(p, t):
            return pltpu.make_async_remote_copy(
                src_ref=kv_send.at[t], dst_ref=kv_recv.at[p],
                send_sem=kv_send_sems.at[t], recv_sem=kv_recv_sems.at[p],
                device_id=(t,), device_id_type=_DeviceIdType.MESH)

        for p in range(N_SRC):
            @pl.when(my == p)
            def _(p=p):
                def _pack(t, kv):
                    _kvst_desc(t, kv, k_ref).wait()
                    return kvst[t, kv].reshape(B, HD, SKV_LOC).astype(
                        jnp.bfloat16)

                for t in _KV_ORDER[p]:
                    kv_send[t, 0] = _pack(t, 0)
                    kv_send[t, 1] = _pack(t, 1)
                    _kv_send_desc(p, t).start()
                kv_recv[p, 0] = _pack(p, 0)
                kv_recv[p, 1] = _pack(p, 1)

        wqtb = (xww_v[2 * SQ:2 * SQ + HD, :] * 0.125).astype(jnp.bfloat16)
        qs = []
        for b in range(B):
            xb = xww_v[b * SQ:(b + 1) * SQ, :].astype(jnp.bfloat16)
            qs.append(lax.dot_general(xb, wqtb, (((1,), (1,)), ((), ())),
                                      preferred_element_type=jnp.float32
                                      ).astype(jnp.bfloat16))

        iota_i = lax.broadcasted_iota(jnp.int32, (SQ, SKV_LOC), 0)
        iota_j = lax.broadcasted_iota(jnp.int32, (SQ, SKV_LOC), 1)
        causal = iota_j <= iota_i

        @pl.when(my != 0)
        def _():
            _kv_send_desc(0, 0).wait_recv()

        qhs = [[qs[b][:, h * DH:(h + 1) * DH]
                for h in range(H_LOC)] for b in range(B)]
        s0s = [[lax.dot(qhs[b][h],
                        kv_recv[0, 0, b][h * DH:(h + 1) * DH, :],
                        preferred_element_type=jnp.float32)
                for h in range(H_LOC)] for b in range(B)]

        @pl.when(my != 1)
        def _():
            _kv_send_desc(1, 1).wait_recv()

        def _rs_desc(s, t, b):
            return pltpu.make_async_remote_copy(
                src_ref=qsend.at[t, b], dst_ref=rs_recv.at[s, b],
                send_sem=rs_send_sems.at[t, b], recv_sem=rs_recv_sems.at[s, b],
                device_id=(t,), device_id_type=_DeviceIdType.MESH)

        def _ag_desc(s, t, b):
            return pltpu.make_async_remote_copy(
                src_ref=ag_send.at[b], dst_ref=ag_recv.at[s, b],
                send_sem=ag_send_sems.at[t, b], recv_sem=ag_recv_sems.at[s, b],
                device_id=(t,), device_id_type=_DeviceIdType.MESH)

        wob = xww_v[2 * SQ + HD:, :].astype(jnp.bfloat16)
        for b in range(B):
            k1 = kv_recv[1, 0, b]
            v0 = kv_recv[0, 1, b]
            v1 = kv_recv[1, 1, b]
            ctx_heads = []
            for h in range(H_LOC):
                hs = slice(h * DH, (h + 1) * DH)
                s1 = lax.dot(qhs[b][h], k1[hs, :],
                             preferred_element_type=jnp.float32)
                w0 = jnp.exp(s0s[b][h])
                w1 = jnp.where(causal, jnp.exp(s1), 0.0)
                l = jnp.sum(w0, axis=1, keepdims=True) + jnp.sum(
                    w1, axis=1, keepdims=True)
                c = (lax.dot_general(w0.astype(jnp.bfloat16), v0[hs, :],
                                     (((1,), (1,)), ((), ())),
                                     preferred_element_type=jnp.float32)
                     + lax.dot_general(w1.astype(jnp.bfloat16), v1[hs, :],
                                       (((1,), (1,)), ((), ())),
                                       preferred_element_type=jnp.float32)) / l
                ctx_heads.append(c)
            ctx_b = jnp.concatenate(ctx_heads, axis=1).astype(jnp.bfloat16)
            pout_b = lax.dot(ctx_b, wob, preferred_element_type=jnp.float32)
            out_stage[b] = pout_b
            for t in range(N_DEV):
                qsend[t, b] = pout_b[:, QW * t:QW * (t + 1)].astype(jnp.bfloat16)
            for s in range(N_DEV):
                @pl.when(my == s)
                def _(s=s, b=b):
                    for t in range(N_DEV):
                        if t != s:
                            _rs_desc(s, t, b).start()

        for b in range(B):
            for s in range(N_DEV):
                @pl.when(my == s)
                def _(s=s, b=b):
                    for t in range(N_DEV):
                        if t != s:
                            _rs_desc(t, s, b).wait_recv()
                    q = out_stage[b, :, QW * s:QW * (s + 1)]
                    for t in range(N_DEV):
                        if t != s:
                            q = q + rs_recv[t, b].astype(jnp.float32)
                    out_stage[b, :, QW * s:QW * (s + 1)] = q
                    ag_send[b] = q.astype(jnp.bfloat16)
                    for t in range(N_DEV):
                        if t != s:
                            _ag_desc(s, t, b).start()

        for b in range(B):
            for s in range(N_DEV):
                @pl.when(my != s)
                def _(s=s, b=b):
                    _ag_desc(s, 0, b).wait_recv()
                    out_stage[b, :, QW * s:QW * (s + 1)] = (
                        ag_recv[s, b].astype(jnp.float32))

        out_cp = pltpu.make_async_copy(out_stage, out_ref, out_sem)
        out_cp.start()

        for b in range(B):
            for s in range(N_DEV):
                @pl.when(my == s)
                def _(s=s, b=b):
                    for t in range(N_DEV):
                        if t != s:
                            _rs_desc(s, t, b).wait_send()
                            _ag_desc(s, t, b).wait_send()
        for p in range(N_SRC):
            @pl.when(my == p)
            def _(p=p):
                for t in range(N_DEV):
                    if t != p:
                        _kv_send_desc(p, t).wait_send()
        out_cp.wait()

        @functools.partial(pl.run_scoped, exit_sem=pltpu.SemaphoreType.REGULAR)
        def _(exit_sem):
            for d in range(1, N_DEV):
                _sem_signal(exit_sem, inc=1, device_id=((my + d) % N_DEV,),
                            device_id_type=_DeviceIdType.MESH)
            _sem_wait(exit_sem, N_DEV - 1)

    return pl.pallas_call(
        body,
        out_shape=jax.ShapeDtypeStruct((B, SQ, D_MODEL), jnp.float32),
        in_specs=[
            pl.BlockSpec(memory_space=pltpu.VMEM),
            pl.BlockSpec(memory_space=pl.ANY),
            pl.BlockSpec(memory_space=pl.ANY),
        ],
        out_specs=pl.BlockSpec(memory_space=pl.ANY),
        scratch_shapes=[
            pltpu.VMEM((N_DEV, 2, B, H_LOC, DH, SKV_LOC), jnp.float32),
            pltpu.VMEM((B, SQ, D_MODEL), jnp.float32),
            pltpu.VMEM((N_DEV, 2, B, HD, SKV_LOC), jnp.bfloat16),
            pltpu.VMEM((N_SRC, 2, B, HD, SKV_LOC), jnp.bfloat16),
            pltpu.VMEM((N_DEV, B, SQ, QW), jnp.bfloat16),
            pltpu.VMEM((N_DEV, B, SQ, QW), jnp.bfloat16),
            pltpu.VMEM((B, SQ, QW), jnp.bfloat16),
            pltpu.VMEM((N_DEV, B, SQ, QW), jnp.bfloat16),
            pltpu.SemaphoreType.DMA((N_DEV, 2)),
            pltpu.SemaphoreType.DMA,
            pltpu.SemaphoreType.DMA((N_DEV,)),
            pltpu.SemaphoreType.DMA((N_SRC,)),
            pltpu.SemaphoreType.DMA((N_DEV, B)),
            pltpu.SemaphoreType.DMA((N_DEV, B)),
            pltpu.SemaphoreType.DMA((N_DEV, B)),
            pltpu.SemaphoreType.DMA((N_DEV, B)),
        ],
        compiler_params=_CompilerParams(collective_id=0),
    )(xww, K_t, V_t)
